# baseline (speedup 1.0000x reference)
import functools

import jax
import jax.numpy as jnp
from jax import lax
from jax.experimental import pallas as pl
from jax.experimental.pallas import tpu as pltpu
from jax.experimental.pallas import tpu_sc as plsc

LOOKUP = 101
PAD = (LOOKUP - 1) // 2
NBINS = 512


def _hist_sc_kernel(num_frames, frame_words, num_pix):
  info = plsc.get_sparse_core_info()
  nw = info.num_cores * info.num_subcores
  assert num_frames % nw == 0
  per_w = num_frames // nw
  n_chunks = num_pix // 16

  mesh = plsc.VectorSubcoreMesh(core_axis_name="c", subcore_axis_name="s")

  @functools.partial(
      pl.kernel,
      mesh=mesh,
      out_type=jax.ShapeDtypeStruct((num_frames * NBINS,), jnp.float32),
      compiler_params=pltpu.CompilerParams(
          use_tc_tiling_on_sc=False, needs_layout_passes=False
      ),
      scratch_types=[
          pltpu.VMEM((frame_words,), jnp.int32),
          pltpu.VMEM((NBINS,), jnp.float32),
      ],
  )
  def hist_kernel(frames_hbm, out_hbm, fbuf, hbuf):
    wid = lax.axis_index("s") * info.num_cores + lax.axis_index("c")
    lanes3 = lax.iota(jnp.int32, 16) * 3
    ones = jnp.ones((16,), jnp.float32)
    zero16 = jnp.zeros((16,), jnp.float32)

    def frame_body(k, carry):
      f = wid * per_w + k
      pltpu.sync_copy(frames_hbm.at[pl.ds(f * frame_words, frame_words)], fbuf)
      for j in range(NBINS // 16):
        hbuf[pl.ds(j * 16, 16)] = zero16

      def pix_body(j, carry2):
        base = j * 48
        idx = lanes3 + base
        r = plsc.load_gather(fbuf, [idx])
        g = plsc.load_gather(fbuf, [idx + 1])
        bl = plsc.load_gather(fbuf, [idx + 2])
        bin_ = ((r >> 5) << 6) + ((g >> 5) << 3) + (bl >> 5)
        plsc.addupdate_scatter(hbuf, [bin_], ones)
        return carry2

      lax.fori_loop(0, n_chunks, pix_body, 0, unroll=8)
      pltpu.sync_copy(hbuf, out_hbm.at[pl.ds(f * NBINS, NBINS)])
      return carry

    lax.fori_loop(0, per_w, frame_body, 0)

  return hist_kernel


def _dense_body(hist_ref, histrev_ref, wrev_ref, b_ref, out_ref):
  x = hist_ref[0]
  t = x.shape[0]
  ss = jnp.sum(x * x, axis=1, keepdims=True)
  xn = x / jnp.maximum(jnp.sqrt(ss), 1e-12)
  xr = histrev_ref[0]
  ssr = jnp.sum(xr * xr, axis=1, keepdims=True)
  xnr = xr / jnp.maximum(jnp.sqrt(ssr), 1e-12)
  srev = lax.dot_general(
      xn,
      xnr,
      (((1,), (1,)), ((), ())),
      preferred_element_type=jnp.float32,
  )
  wpad = 256
  pf = jnp.concatenate([jnp.zeros((t, wpad - t), jnp.float32), srev], axis=1)
  rolled = pltpu.roll(pf, 56, 1, stride=1, stride_axis=0)
  g = rolled[:, 5 : 5 + LOOKUP]
  o = lax.dot_general(
      g,
      wrev_ref[...],
      (((1,), (1,)), ((), ())),
      preferred_element_type=jnp.float32,
  )
  out_ref[0] = jnp.maximum(o + b_ref[...], 0.0)


@jax.jit
def kernel(frames, W, b):
  B, T, H, Wd, C = frames.shape
  num_frames = B * T
  num_pix = H * Wd
  frame_words = num_pix * C

  hist_flat = _hist_sc_kernel(num_frames, frame_words, num_pix)(
      frames.reshape(-1)
  )
  hist = hist_flat.reshape(B, T, NBINS)

  odim = W.shape[0]
  out = pl.pallas_call(
      _dense_body,
      out_shape=jax.ShapeDtypeStruct((B, T, odim), jnp.float32),
      grid=(B,),
      in_specs=[
          pl.BlockSpec((1, T, NBINS), lambda i: (i, 0, 0)),
          pl.BlockSpec((1, T, NBINS), lambda i: (i, 0, 0)),
          pl.BlockSpec((odim, LOOKUP), lambda i: (0, 0)),
          pl.BlockSpec((1, odim), lambda i: (0, 0)),
      ],
      out_specs=pl.BlockSpec((1, T, odim), lambda i: (i, 0, 0)),
  )(hist, hist[:, ::-1, :], W[:, ::-1], b.reshape(1, odim))
  return out


# speedup vs baseline: 1.0030x; 1.0030x over previous
import functools

import jax
import jax.numpy as jnp
from jax import lax
from jax.experimental import pallas as pl
from jax.experimental.pallas import tpu as pltpu
from jax.experimental.pallas import tpu_sc as plsc

LOOKUP = 101
PAD = (LOOKUP - 1) // 2
NBINS = 512


def _hist_sc_kernel(num_frames, frame_words, num_pix):
  info = plsc.get_sparse_core_info()
  nw = info.num_cores * info.num_subcores
  assert num_frames % nw == 0
  per_w = num_frames // nw
  n_chunks = num_pix // 16

  mesh = plsc.VectorSubcoreMesh(core_axis_name="c", subcore_axis_name="s")

  @functools.partial(
      pl.kernel,
      mesh=mesh,
      out_type=jax.ShapeDtypeStruct((num_frames * NBINS,), jnp.float32),
      compiler_params=pltpu.CompilerParams(
          use_tc_tiling_on_sc=False, needs_layout_passes=False
      ),
      scratch_types=[
          pltpu.VMEM((frame_words,), jnp.int32),
          pltpu.VMEM((NBINS,), jnp.float32),
      ],
  )
  def hist_kernel(frames_hbm, out_hbm, fbuf, hbuf):
    wid = lax.axis_index("s") * info.num_cores + lax.axis_index("c")
    lanes3 = lax.iota(jnp.int32, 16) * 3
    ones = jnp.ones((16,), jnp.float32)
    zero16 = jnp.zeros((16,), jnp.float32)

    def frame_body(k, carry):
      f = wid * per_w + k
      pltpu.sync_copy(frames_hbm.at[pl.ds(f * frame_words, frame_words)], fbuf)
      for j in range(NBINS // 16):
        hbuf[pl.ds(j * 16, 16)] = zero16

      def pix_body(j, carry2):
        base = j * 48
        idx = lanes3 + base
        r = plsc.load_gather(fbuf, [idx])
        g = plsc.load_gather(fbuf, [idx + 1])
        bl = plsc.load_gather(fbuf, [idx + 2])
        bin_ = ((r >> 5) << 6) + ((g >> 5) << 3) + (bl >> 5)
        plsc.addupdate_scatter(hbuf, [bin_], ones)
        return carry2

      lax.fori_loop(0, n_chunks, pix_body, 0, unroll=8)
      pltpu.sync_copy(hbuf, out_hbm.at[pl.ds(f * NBINS, NBINS)])
      return carry

    lax.fori_loop(0, per_w, frame_body, 0)

  return hist_kernel


def _dense_body(hist_ref, w_ref, b_ref, out_ref):
  x = hist_ref[0]
  t = x.shape[0]
  ss = jnp.sum(x * x, axis=1, keepdims=True)
  xn = x / jnp.maximum(jnp.sqrt(ss), 1e-12)
  ii = lax.broadcasted_iota(jnp.int32, (t, t), 0)
  kk = lax.broadcasted_iota(jnp.int32, (t, t), 1)
  revp = jnp.where(ii + kk == t - 1, 1.0, 0.0).astype(jnp.float32)
  xnr = lax.dot_general(
      revp, xn, (((1,), (0,)), ((), ())), preferred_element_type=jnp.float32
  )
  srev = lax.dot_general(
      xn,
      xnr,
      (((1,), (1,)), ((), ())),
      preferred_element_type=jnp.float32,
  )
  wpad = 256
  pf = jnp.concatenate([jnp.zeros((t, wpad - t), jnp.float32), srev], axis=1)
  rolled = pltpu.roll(pf, 56, 1, stride=1, stride_axis=0)
  g = rolled[:, 5 : 5 + LOOKUP]
  o = lax.dot_general(
      g,
      w_ref[...],
      (((1,), (1,)), ((), ())),
      preferred_element_type=jnp.float32,
  )
  out_ref[0] = jnp.maximum(o + b_ref[...], 0.0)


@jax.jit
def kernel(frames, W, b):
  B, T, H, Wd, C = frames.shape
  num_frames = B * T
  num_pix = H * Wd
  frame_words = num_pix * C

  hist_flat = _hist_sc_kernel(num_frames, frame_words, num_pix)(
      frames.reshape(-1)
  )
  hist = hist_flat.reshape(B, T, NBINS)

  odim = W.shape[0]
  out = pl.pallas_call(
      _dense_body,
      out_shape=jax.ShapeDtypeStruct((B, T, odim), jnp.float32),
      grid=(B,),
      in_specs=[
          pl.BlockSpec((1, T, NBINS), lambda i: (i, 0, 0)),
          pl.BlockSpec((odim, LOOKUP), lambda i: (0, 0)),
          pl.BlockSpec((1, odim), lambda i: (0, 0)),
      ],
      out_specs=pl.BlockSpec((1, T, odim), lambda i: (i, 0, 0)),
  )(hist, W[:, ::-1], b.reshape(1, odim))
  return out


# speedup vs baseline: 27.9171x; 27.8340x over previous
import functools

import jax
import jax.numpy as jnp
from jax import lax
from jax.experimental import pallas as pl
from jax.experimental.pallas import tpu as pltpu
from jax.experimental.pallas import tpu_sc as plsc

LOOKUP = 101
PAD = (LOOKUP - 1) // 2
NBINS = 512


def _hist_sc_kernel(num_frames, num_pix):
  info = plsc.get_sparse_core_info()
  nw = info.num_cores * info.num_subcores
  assert num_frames % nw == 0
  per_w = num_frames // nw
  n_chunks = num_pix // 16

  mesh = plsc.VectorSubcoreMesh(core_axis_name="c", subcore_axis_name="s")

  @functools.partial(
      pl.kernel,
      mesh=mesh,
      out_type=jax.ShapeDtypeStruct((num_frames * NBINS,), jnp.float32),
      compiler_params=pltpu.CompilerParams(
          use_tc_tiling_on_sc=False, needs_layout_passes=False
      ),
      scratch_types=[
          pltpu.VMEM((num_pix,), jnp.int32),
          pltpu.VMEM((num_pix,), jnp.int32),
          pltpu.VMEM((num_pix,), jnp.int32),
          pltpu.VMEM((NBINS,), jnp.float32),
      ],
  )
  def hist_kernel(rp_hbm, gp_hbm, bp_hbm, out_hbm, rbuf, gbuf, bbuf, hbuf):
    wid = lax.axis_index("s") * info.num_cores + lax.axis_index("c")
    ones = jnp.ones((16,), jnp.float32)
    zero16 = jnp.zeros((16,), jnp.float32)

    def frame_body(k, carry):
      f = wid * per_w + k
      base = f * num_pix
      pltpu.sync_copy(rp_hbm.at[pl.ds(base, num_pix)], rbuf)
      pltpu.sync_copy(gp_hbm.at[pl.ds(base, num_pix)], gbuf)
      pltpu.sync_copy(bp_hbm.at[pl.ds(base, num_pix)], bbuf)
      for j in range(NBINS // 16):
        hbuf[pl.ds(j * 16, 16)] = zero16

      def pix_body(j, carry2):
        o = j * 16
        r = rbuf[pl.ds(o, 16)]
        g = gbuf[pl.ds(o, 16)]
        bl = bbuf[pl.ds(o, 16)]
        bin_ = ((r >> 5) << 6) + ((g >> 5) << 3) + (bl >> 5)
        plsc.addupdate_scatter(hbuf, [bin_], ones)
        return carry2

      lax.fori_loop(0, n_chunks, pix_body, 0, unroll=8)
      pltpu.sync_copy(hbuf, out_hbm.at[pl.ds(f * NBINS, NBINS)])
      return carry

    lax.fori_loop(0, per_w, frame_body, 0)

  return hist_kernel


def _dense_body(hist_ref, w_ref, b_ref, out_ref):
  x = hist_ref[0]
  t = x.shape[0]
  ss = jnp.sum(x * x, axis=1, keepdims=True)
  xn = x / jnp.maximum(jnp.sqrt(ss), 1e-12)
  ii = lax.broadcasted_iota(jnp.int32, (t, t), 0)
  kk = lax.broadcasted_iota(jnp.int32, (t, t), 1)
  revp = jnp.where(ii + kk == t - 1, 1.0, 0.0).astype(jnp.float32)
  xnr = lax.dot_general(
      revp, xn, (((1,), (0,)), ((), ())), preferred_element_type=jnp.float32
  )
  srev = lax.dot_general(
      xn,
      xnr,
      (((1,), (1,)), ((), ())),
      preferred_element_type=jnp.float32,
  )
  wpad = 256
  pf = jnp.concatenate([jnp.zeros((t, wpad - t), jnp.float32), srev], axis=1)
  rolled = pltpu.roll(pf, 56, 1, stride=1, stride_axis=0)
  g = rolled[:, 5 : 5 + LOOKUP]
  o = lax.dot_general(
      g,
      w_ref[...],
      (((1,), (1,)), ((), ())),
      preferred_element_type=jnp.float32,
  )
  out_ref[0] = jnp.maximum(o + b_ref[...], 0.0)


@jax.jit
def kernel(frames, W, b):
  B, T, H, Wd, C = frames.shape
  num_frames = B * T
  num_pix = H * Wd

  rp = frames[..., 0].reshape(-1)
  gp = frames[..., 1].reshape(-1)
  bp = frames[..., 2].reshape(-1)
  hist_flat = _hist_sc_kernel(num_frames, num_pix)(rp, gp, bp)
  hist = hist_flat.reshape(B, T, NBINS)

  odim = W.shape[0]
  out = pl.pallas_call(
      _dense_body,
      out_shape=jax.ShapeDtypeStruct((B, T, odim), jnp.float32),
      grid=(B,),
      in_specs=[
          pl.BlockSpec((1, T, NBINS), lambda i: (i, 0, 0)),
          pl.BlockSpec((odim, LOOKUP), lambda i: (0, 0)),
          pl.BlockSpec((1, odim), lambda i: (0, 0)),
      ],
      out_specs=pl.BlockSpec((1, T, odim), lambda i: (i, 0, 0)),
  )(hist, W[:, ::-1], b.reshape(1, odim))
  return out


# speedup vs baseline: 82.9499x; 2.9713x over previous
import functools

import jax
import jax.numpy as jnp
from jax import lax
from jax.experimental import pallas as pl
from jax.experimental.pallas import tpu as pltpu
from jax.experimental.pallas import tpu_sc as plsc

LOOKUP = 101
PAD = (LOOKUP - 1) // 2
NBINS = 512
TPADDED = 256


def _bins_body(x_ref, out_ref):
  x = x_ref[0]
  r = x[:, 0]
  g = x[:, 1]
  b = x[:, 2]
  bins = ((r >> 5) << 6) + ((g >> 5) << 3) + (b >> 5)
  out_ref[0, :, :, : bins.shape[-1]] = bins


def _bins_kernel(frames_t, hb):
  B, H, C, Wd, T = frames_t.shape
  return pl.pallas_call(
      _bins_body,
      out_shape=jax.ShapeDtypeStruct((B, H, Wd, TPADDED), jnp.int32),
      grid=(B, H // hb),
      in_specs=[
          pl.BlockSpec((1, hb, C, Wd, T), lambda i, j: (i, j, 0, 0, 0)),
      ],
      out_specs=pl.BlockSpec((1, hb, Wd, TPADDED), lambda i, j: (i, j, 0, 0)),
  )(frames_t)


def _hist_sc_kernel(B, T, H, Wd):
  info = plsc.get_sparse_core_info()
  nw = info.num_cores * info.num_subcores
  halves = nw // B
  assert H % halves == 0
  rows_per_w = H // halves
  hwords = T * NBINS
  n_full = T // 16
  mesh = plsc.VectorSubcoreMesh(core_axis_name="c", subcore_axis_name="s")

  @functools.partial(
      pl.kernel,
      mesh=mesh,
      out_type=jax.ShapeDtypeStruct((nw * hwords,), jnp.float32),
      compiler_params=pltpu.CompilerParams(
          use_tc_tiling_on_sc=False, needs_layout_passes=False
      ),
      scratch_types=[
          pltpu.VMEM((Wd * TPADDED,), jnp.int32),
          pltpu.VMEM((hwords,), jnp.float32),
      ],
  )
  def hist_kernel(bins_hbm, out_hbm, slab, hist):
    wid = lax.axis_index("s") * info.num_cores + lax.axis_index("c")
    b = wid // halves
    half = wid % halves
    ones = jnp.ones((16,), jnp.float32)
    zero16 = jnp.zeros((16,), jnp.float32)
    tvec = lax.iota(jnp.int32, 16) * NBINS
    tail_mask = lax.iota(jnp.int32, 16) < (T - n_full * 16)

    def zero_body(j, carry):
      hist[pl.ds(j * 16, 16)] = zero16
      return carry

    lax.fori_loop(0, hwords // 16, zero_body, 0, unroll=16)

    row_words = Wd * TPADDED

    def slab_body(i, carry):
      h = half * rows_per_w + i
      base = (b * H + h) * row_words
      pltpu.sync_copy(bins_hbm.at[pl.ds(base, row_words)], slab)

      def pix_body(w, carry2):
        o = w * TPADDED

        def chunk_body(j, carry3):
          v = slab[pl.ds(o + j * 16, 16)]
          idx = v + tvec + (j * (16 * NBINS))
          plsc.addupdate_scatter(hist, [idx], ones)
          return carry3

        lax.fori_loop(0, n_full, chunk_body, 0, unroll=12)
        v = slab[pl.ds(o + n_full * 16, 16)]
        idx = v + tvec + (n_full * (16 * NBINS))
        plsc.addupdate_scatter(hist, [idx], ones, mask=tail_mask)
        return carry2

      lax.fori_loop(0, Wd, pix_body, 0)
      return carry

    lax.fori_loop(0, rows_per_w, slab_body, 0)
    pltpu.sync_copy(hist, out_hbm.at[pl.ds(wid * hwords, hwords)])

  return hist_kernel


def _dense_body(h_ref, w_ref, b_ref, out_ref):
  x = h_ref[0, 0] + h_ref[0, 1]
  t = x.shape[0]
  ss = jnp.sum(x * x, axis=1, keepdims=True)
  xn = x / jnp.maximum(jnp.sqrt(ss), 1e-12)
  ii = lax.broadcasted_iota(jnp.int32, (t, t), 0)
  kk = lax.broadcasted_iota(jnp.int32, (t, t), 1)
  revp = jnp.where(ii + kk == t - 1, 1.0, 0.0).astype(jnp.float32)
  xnr = lax.dot_general(
      revp, xn, (((1,), (0,)), ((), ())), preferred_element_type=jnp.float32
  )
  srev = lax.dot_general(
      xn,
      xnr,
      (((1,), (1,)), ((), ())),
      preferred_element_type=jnp.float32,
  )
  wpad = 256
  pf = jnp.concatenate([jnp.zeros((t, wpad - t), jnp.float32), srev], axis=1)
  rolled = pltpu.roll(pf, 56, 1, stride=1, stride_axis=0)
  g = rolled[:, 5 : 5 + LOOKUP]
  o = lax.dot_general(
      g,
      w_ref[...],
      (((1,), (1,)), ((), ())),
      preferred_element_type=jnp.float32,
  )
  out_ref[0] = jnp.maximum(o + b_ref[...], 0.0)


@jax.jit
def kernel(frames, W, b):
  B, T, H, Wd, C = frames.shape

  frames_t = jnp.transpose(frames, (0, 2, 4, 3, 1))
  bins = _bins_kernel(frames_t, hb=12)
  partials = _hist_sc_kernel(B, T, H, Wd)(bins.reshape(-1))
  halves = 32 // B
  hist = partials.reshape(B, halves, T, NBINS)

  odim = W.shape[0]
  out = pl.pallas_call(
      _dense_body,
      out_shape=jax.ShapeDtypeStruct((B, T, odim), jnp.float32),
      grid=(B,),
      in_specs=[
          pl.BlockSpec((1, halves, T, NBINS), lambda i: (i, 0, 0, 0)),
          pl.BlockSpec((odim, LOOKUP), lambda i: (0, 0)),
          pl.BlockSpec((1, odim), lambda i: (0, 0)),
      ],
      out_specs=pl.BlockSpec((1, T, odim), lambda i: (i, 0, 0)),
  )(hist, W[:, ::-1], b.reshape(1, odim))
  return out


# speedup vs baseline: 87.0151x; 1.0490x over previous
import functools

import jax
import jax.numpy as jnp
from jax import lax
from jax.experimental import pallas as pl
from jax.experimental.pallas import tpu as pltpu
from jax.experimental.pallas import tpu_sc as plsc

LOOKUP = 101
PAD = (LOOKUP - 1) // 2
NBINS = 512
TPADDED = 256


def _bins_body(x_ref, out_ref):
  x = x_ref[0]
  r = x[:, 0]
  g = x[:, 1]
  b = x[:, 2]
  bins = ((r >> 5) << 6) + ((g >> 5) << 3) + (b >> 5)
  out_ref[0, :, :, : bins.shape[-1]] = bins


def _bins_kernel(frames_t, hb):
  B, H, C, Wd, T = frames_t.shape
  return pl.pallas_call(
      _bins_body,
      out_shape=jax.ShapeDtypeStruct((B, H, Wd, TPADDED), jnp.int32),
      grid=(B, H // hb),
      in_specs=[
          pl.BlockSpec((1, hb, C, Wd, T), lambda i, j: (i, j, 0, 0, 0)),
      ],
      out_specs=pl.BlockSpec((1, hb, Wd, TPADDED), lambda i, j: (i, j, 0, 0)),
  )(frames_t)


TCOLS = 208


def _hist_sc_kernel(B, T, H, Wd):
  info = plsc.get_sparse_core_info()
  nw = info.num_cores * info.num_subcores
  halves = nw // B
  assert H % halves == 0
  rows_per_w = H // halves
  n_full = T // 16
  half_w = Wd // 2
  halfw_words = half_w * TPADDED
  n_steps = rows_per_w * 2
  mesh = plsc.VectorSubcoreMesh(core_axis_name="c", subcore_axis_name="s")

  @functools.partial(
      pl.kernel,
      mesh=mesh,
      out_type=jax.ShapeDtypeStruct((nw, NBINS, TCOLS), jnp.float32),
      compiler_params=pltpu.CompilerParams(
          use_tc_tiling_on_sc=False, needs_layout_passes=False
      ),
      scratch_types=[
          pltpu.VMEM((2, halfw_words), jnp.int32),
          pltpu.VMEM((NBINS, TCOLS), jnp.float32),
          pltpu.SemaphoreType.DMA,
          pltpu.SemaphoreType.DMA,
      ],
  )
  def hist_kernel(bins_hbm, out_hbm, slab, hist, sem0, sem1):
    sems = (sem0, sem1)
    wid = lax.axis_index("s") * info.num_cores + lax.axis_index("c")
    b = wid // halves
    half = wid % halves
    ones = jnp.ones((16,), jnp.float32)
    zero16 = jnp.zeros((16,), jnp.float32)
    tconst = [lax.iota(jnp.int32, 16) + j * 16 for j in range(n_full + 1)]
    tail_mask = lax.iota(jnp.int32, 16) < (T - n_full * 16)

    def zero_body(j, carry):
      hist[j, pl.ds(0, 16)] = zero16
      for k in range(1, TCOLS // 16):
        hist[j, pl.ds(k * 16, 16)] = zero16
      return carry

    lax.fori_loop(0, NBINS, zero_body, 0)

    def src_slice(i):
      h = half * rows_per_w + i // 2
      base = ((b * H + h) * Wd + (i % 2) * half_w) * TPADDED
      return bins_hbm.at[pl.ds(base, halfw_words)]

    pltpu.async_copy(src_slice(0), slab.at[0], sems[0])
    pltpu.async_copy(src_slice(1), slab.at[1], sems[1])

    def step_body(g, carry):
      for bi in range(2):
        i = g * 2 + bi
        pltpu.make_async_copy(src_slice(0), slab.at[bi], sems[bi]).wait()

        def pix_body(w, carry2):
          o = w * TPADDED
          for j in range(n_full):
            v = slab[bi, pl.ds(o + j * 16, 16)]
            plsc.addupdate_scatter(hist, [v, tconst[j]], ones)
          v = slab[bi, pl.ds(o + n_full * 16, 16)]
          plsc.addupdate_scatter(hist, [v, tconst[n_full]], ones,
                                 mask=tail_mask)
          return carry2

        lax.fori_loop(0, half_w, pix_body, 0)

        @pl.when(i + 2 < n_steps)
        def _():
          pltpu.async_copy(src_slice(i + 2), slab.at[bi], sems[bi])

      return carry

    lax.fori_loop(0, n_steps // 2, step_body, 0)
    pltpu.sync_copy(hist, out_hbm.at[wid])

  return hist_kernel


def _dense_body(h_ref, w_ref, b_ref, out_ref, *, t):
  xt = h_ref[0, 0, :, :t] + h_ref[0, 1, :, :t]
  ss = jnp.sum(xt * xt, axis=0, keepdims=True)
  xnt = xt / jnp.maximum(jnp.sqrt(ss), 1e-12)
  ii = lax.broadcasted_iota(jnp.int32, (t, t), 0)
  kk = lax.broadcasted_iota(jnp.int32, (t, t), 1)
  revp = jnp.where(ii + kk == t - 1, 1.0, 0.0).astype(jnp.float32)
  xnrt = lax.dot_general(
      xnt, revp, (((1,), (0,)), ((), ())), preferred_element_type=jnp.float32
  )
  srev = lax.dot_general(
      xnt,
      xnrt,
      (((0,), (0,)), ((), ())),
      preferred_element_type=jnp.float32,
  )
  wpad = 256
  pf = jnp.concatenate([jnp.zeros((t, wpad - t), jnp.float32), srev], axis=1)
  rolled = pltpu.roll(pf, 56, 1, stride=1, stride_axis=0)
  g = rolled[:, 5 : 5 + LOOKUP]
  o = lax.dot_general(
      g,
      w_ref[...],
      (((1,), (1,)), ((), ())),
      preferred_element_type=jnp.float32,
  )
  out_ref[0] = jnp.maximum(o + b_ref[...], 0.0)


@jax.jit
def kernel(frames, W, b):
  B, T, H, Wd, C = frames.shape

  frames_t = jnp.transpose(frames, (0, 2, 4, 3, 1))
  bins = _bins_kernel(frames_t, hb=12)
  partials = _hist_sc_kernel(B, T, H, Wd)(bins.reshape(-1))
  halves = 32 // B
  hist = partials.reshape(B, halves, NBINS, TCOLS)

  odim = W.shape[0]
  out = pl.pallas_call(
      functools.partial(_dense_body, t=T),
      out_shape=jax.ShapeDtypeStruct((B, T, odim), jnp.float32),
      grid=(B,),
      in_specs=[
          pl.BlockSpec((1, halves, NBINS, TCOLS), lambda i: (i, 0, 0, 0)),
          pl.BlockSpec((odim, LOOKUP), lambda i: (0, 0)),
          pl.BlockSpec((1, odim), lambda i: (0, 0)),
      ],
      out_specs=pl.BlockSpec((1, T, odim), lambda i: (i, 0, 0)),
  )(hist, W[:, ::-1], b.reshape(1, odim))
  return out


# speedup vs baseline: 87.4390x; 1.0049x over previous
import functools

import jax
import jax.numpy as jnp
from jax import lax
from jax.experimental import pallas as pl
from jax.experimental.pallas import tpu as pltpu
from jax.experimental.pallas import tpu_sc as plsc

LOOKUP = 101
PAD = (LOOKUP - 1) // 2
NBINS = 512
TPADDED = 256


def _bins_body(x_ref, out_ref):
  x = x_ref[0]
  r = x[:, 0]
  g = x[:, 1]
  b = x[:, 2]
  bins = ((r >> 5) << 6) + ((g >> 5) << 3) + (b >> 5)
  out_ref[0, :, :, : bins.shape[-1]] = bins


def _bins_kernel(frames_t, hb):
  B, H, C, Wd, T = frames_t.shape
  return pl.pallas_call(
      _bins_body,
      out_shape=jax.ShapeDtypeStruct((B, H, Wd, TPADDED), jnp.int32),
      grid=(B, H // hb),
      in_specs=[
          pl.BlockSpec((1, hb, C, Wd, T), lambda i, j: (i, j, 0, 0, 0)),
      ],
      out_specs=pl.BlockSpec((1, hb, Wd, TPADDED), lambda i, j: (i, j, 0, 0)),
  )(frames_t)


TCOLS = 208


def _hist_sc_kernel(B, T, H, Wd):
  info = plsc.get_sparse_core_info()
  nw = info.num_cores * info.num_subcores
  halves = nw // B
  assert H % halves == 0
  rows_per_w = H // halves
  n_full = T // 16
  half_w = Wd // 2
  halfw_words = half_w * TPADDED
  n_steps = rows_per_w * 2
  mesh = plsc.VectorSubcoreMesh(core_axis_name="c", subcore_axis_name="s")

  @functools.partial(
      pl.kernel,
      mesh=mesh,
      out_type=jax.ShapeDtypeStruct((nw, NBINS, TPADDED), jnp.float32),
      compiler_params=pltpu.CompilerParams(
          use_tc_tiling_on_sc=False, needs_layout_passes=False
      ),
      scratch_types=[
          pltpu.VMEM((2, halfw_words), jnp.int32),
          pltpu.VMEM((NBINS, TCOLS), jnp.float32),
          pltpu.SemaphoreType.DMA,
          pltpu.SemaphoreType.DMA,
      ],
  )
  def hist_kernel(bins_hbm, out_hbm, slab, hist, sem0, sem1):
    sems = (sem0, sem1)
    wid = lax.axis_index("s") * info.num_cores + lax.axis_index("c")
    b = wid // halves
    half = wid % halves
    ones = jnp.ones((16,), jnp.float32)
    zero16 = jnp.zeros((16,), jnp.float32)
    tconst = [lax.iota(jnp.int32, 16) + j * 16 for j in range(n_full + 1)]
    tail_mask = lax.iota(jnp.int32, 16) < (T - n_full * 16)

    def zero_body(j, carry):
      hist[j, pl.ds(0, 16)] = zero16
      for k in range(1, TCOLS // 16):
        hist[j, pl.ds(k * 16, 16)] = zero16
      return carry

    lax.fori_loop(0, NBINS, zero_body, 0)

    def src_slice(i):
      h = half * rows_per_w + i // 2
      base = ((b * H + h) * Wd + (i % 2) * half_w) * TPADDED
      return bins_hbm.at[pl.ds(base, halfw_words)]

    pltpu.async_copy(src_slice(0), slab.at[0], sems[0])
    pltpu.async_copy(src_slice(1), slab.at[1], sems[1])

    def step_body(g, carry):
      for bi in range(2):
        i = g * 2 + bi
        pltpu.make_async_copy(src_slice(0), slab.at[bi], sems[bi]).wait()

        def pix_body(w, carry2):
          o = w * TPADDED
          for j in range(n_full):
            v = slab[bi, pl.ds(o + j * 16, 16)]
            plsc.addupdate_scatter(hist, [v, tconst[j]], ones)
          v = slab[bi, pl.ds(o + n_full * 16, 16)]
          plsc.addupdate_scatter(hist, [v, tconst[n_full]], ones,
                                 mask=tail_mask)
          return carry2

        lax.fori_loop(0, half_w, pix_body, 0)

        @pl.when(i + 2 < n_steps)
        def _():
          pltpu.async_copy(src_slice(i + 2), slab.at[bi], sems[bi])

      return carry

    lax.fori_loop(0, n_steps // 2, step_body, 0)
    pltpu.sync_copy(hist, out_hbm.at[wid, :, pl.ds(0, TCOLS)])

  return hist_kernel


def _dense_body(h_ref, w_ref, b_ref, out_ref, *, t):
  xt = h_ref[0, 0, :, :t] + h_ref[0, 1, :, :t]
  ss = jnp.sum(xt * xt, axis=0, keepdims=True)
  xnt = xt / jnp.maximum(jnp.sqrt(ss), 1e-12)
  ii = lax.broadcasted_iota(jnp.int32, (t, t), 0)
  kk = lax.broadcasted_iota(jnp.int32, (t, t), 1)
  revp = jnp.where(ii + kk == t - 1, 1.0, 0.0).astype(jnp.float32)
  xnrt = lax.dot_general(
      xnt, revp, (((1,), (0,)), ((), ())), preferred_element_type=jnp.float32
  )
  srev = lax.dot_general(
      xnt,
      xnrt,
      (((0,), (0,)), ((), ())),
      preferred_element_type=jnp.float32,
  )
  wpad = 256
  pf = jnp.concatenate([jnp.zeros((t, wpad - t), jnp.float32), srev], axis=1)
  rolled = pltpu.roll(pf, 56, 1, stride=1, stride_axis=0)
  g = rolled[:, 5 : 5 + LOOKUP]
  o = lax.dot_general(
      g,
      w_ref[...],
      (((1,), (1,)), ((), ())),
      preferred_element_type=jnp.float32,
  )
  out_ref[0] = jnp.maximum(o + b_ref[...], 0.0)


@jax.jit
def kernel(frames, W, b):
  B, T, H, Wd, C = frames.shape

  frames_t = jnp.transpose(frames, (0, 2, 4, 3, 1))
  bins = _bins_kernel(frames_t, hb=12)
  partials = _hist_sc_kernel(B, T, H, Wd)(bins.reshape(-1))
  halves = 32 // B
  hist = partials.reshape(B, halves, NBINS, TPADDED)

  odim = W.shape[0]
  out = pl.pallas_call(
      functools.partial(_dense_body, t=T),
      out_shape=jax.ShapeDtypeStruct((B, T, odim), jnp.float32),
      grid=(B,),
      in_specs=[
          pl.BlockSpec((1, halves, NBINS, TPADDED), lambda i: (i, 0, 0, 0)),
          pl.BlockSpec((odim, LOOKUP), lambda i: (0, 0)),
          pl.BlockSpec((1, odim), lambda i: (0, 0)),
      ],
      out_specs=pl.BlockSpec((1, T, odim), lambda i: (i, 0, 0)),
  )(hist, W[:, ::-1], b.reshape(1, odim))
  return out


# speedup vs baseline: 123.5799x; 1.4133x over previous
import functools

import jax
import jax.numpy as jnp
from jax import lax
from jax.experimental import pallas as pl
from jax.experimental.pallas import tpu as pltpu
from jax.experimental.pallas import tpu_sc as plsc

LOOKUP = 101
PAD = (LOOKUP - 1) // 2
NBINS = 512
TPADDED = 256


def _bins_body(x_ref, out_ref):
  x = x_ref[0]
  r = x[:, 0]
  g = x[:, 1]
  b = x[:, 2]
  bins = ((r >> 5) << 6) + ((g >> 5) << 3) + (b >> 5)
  out_ref[0, :, :, : bins.shape[-1]] = bins


def _bins_kernel(frames_t, hb):
  B, H, C, Wd, T = frames_t.shape
  return pl.pallas_call(
      _bins_body,
      out_shape=jax.ShapeDtypeStruct((B, H, Wd, TPADDED), jnp.int32),
      grid=(B, H // hb),
      in_specs=[
          pl.BlockSpec((1, hb, C, Wd, T), lambda i, j: (i, j, 0, 0, 0)),
      ],
      out_specs=pl.BlockSpec((1, hb, Wd, TPADDED), lambda i, j: (i, j, 0, 0)),
  )(frames_t)


TCOLS = 208


def _hist_sc_kernel(B, T, H, Wd):
  info = plsc.get_sparse_core_info()
  nw = info.num_cores * info.num_subcores
  halves = nw // B
  assert H % halves == 0
  rows_per_w = H // halves
  n_full = T // 16
  half_w = Wd // 2
  halfw_words = half_w * TPADDED
  n_steps = rows_per_w * 2
  mesh = plsc.VectorSubcoreMesh(core_axis_name="c", subcore_axis_name="s")

  @functools.partial(
      pl.kernel,
      mesh=mesh,
      out_type=jax.ShapeDtypeStruct((nw, NBINS, TPADDED), jnp.float32),
      compiler_params=pltpu.CompilerParams(
          use_tc_tiling_on_sc=False, needs_layout_passes=False
      ),
      scratch_types=[
          pltpu.VMEM((2, halfw_words), jnp.int32),
          pltpu.VMEM((NBINS, TCOLS), jnp.float32),
          pltpu.SemaphoreType.DMA,
          pltpu.SemaphoreType.DMA,
      ],
  )
  def hist_kernel(bins_hbm, out_hbm, slab, hist, sem0, sem1):
    sems = (sem0, sem1)
    wid = lax.axis_index("s") * info.num_cores + lax.axis_index("c")
    b = wid // halves
    half = wid % halves
    ones = jnp.ones((16,), jnp.float32)
    zero16 = jnp.zeros((16,), jnp.float32)
    tconst = [lax.iota(jnp.int32, 16) + j * 16 for j in range(n_full + 1)]
    tail_mask = lax.iota(jnp.int32, 16) < (T - n_full * 16)

    def zero_body(j, carry):
      hist[j, pl.ds(0, 16)] = zero16
      for k in range(1, TCOLS // 16):
        hist[j, pl.ds(k * 16, 16)] = zero16
      return carry

    lax.fori_loop(0, NBINS, zero_body, 0)

    def src_slice(i):
      h = half * rows_per_w + i // 2
      base = ((b * H + h) * Wd + (i % 2) * half_w) * TPADDED
      return bins_hbm.at[pl.ds(base, halfw_words)]

    pltpu.async_copy(src_slice(0), slab.at[0], sems[0])
    pltpu.async_copy(src_slice(1), slab.at[1], sems[1])

    def step_body(g, carry):
      for bi in range(2):
        i = g * 2 + bi
        pltpu.make_async_copy(src_slice(0), slab.at[bi], sems[bi]).wait()

        @plsc.parallel_loop(0, half_w, 1, unroll=2)
        def pix_body(w):
          o = w * TPADDED
          for j in range(n_full):
            v = slab[bi, pl.ds(o + j * 16, 16)]
            plsc.addupdate_scatter(hist, [v, tconst[j]], ones)
          v = slab[bi, pl.ds(o + n_full * 16, 16)]
          plsc.addupdate_scatter(hist, [v, tconst[n_full]], ones,
                                 mask=tail_mask)

        @pl.when(i + 2 < n_steps)
        def _():
          pltpu.async_copy(src_slice(i + 2), slab.at[bi], sems[bi])

      return carry

    lax.fori_loop(0, n_steps // 2, step_body, 0)
    pltpu.sync_copy(hist, out_hbm.at[wid, :, pl.ds(0, TCOLS)])

  return hist_kernel


def _dense_body(h_ref, w_ref, b_ref, out_ref, *, t):
  xt = h_ref[0, 0, :, :t] + h_ref[0, 1, :, :t]
  ss = jnp.sum(xt * xt, axis=0, keepdims=True)
  xnt = xt / jnp.maximum(jnp.sqrt(ss), 1e-12)
  ii = lax.broadcasted_iota(jnp.int32, (t, t), 0)
  kk = lax.broadcasted_iota(jnp.int32, (t, t), 1)
  revp = jnp.where(ii + kk == t - 1, 1.0, 0.0).astype(jnp.float32)
  xnrt = lax.dot_general(
      xnt, revp, (((1,), (0,)), ((), ())), preferred_element_type=jnp.float32
  )
  srev = lax.dot_general(
      xnt,
      xnrt,
      (((0,), (0,)), ((), ())),
      preferred_element_type=jnp.float32,
  )
  wpad = 256
  pf = jnp.concatenate([jnp.zeros((t, wpad - t), jnp.float32), srev], axis=1)
  rolled = pltpu.roll(pf, 56, 1, stride=1, stride_axis=0)
  g = rolled[:, 5 : 5 + LOOKUP]
  o = lax.dot_general(
      g,
      w_ref[...],
      (((1,), (1,)), ((), ())),
      preferred_element_type=jnp.float32,
  )
  out_ref[0] = jnp.maximum(o + b_ref[...], 0.0)


@jax.jit
def kernel(frames, W, b):
  B, T, H, Wd, C = frames.shape

  frames_t = jnp.transpose(frames, (0, 2, 4, 3, 1))
  bins = _bins_kernel(frames_t, hb=12)
  partials = _hist_sc_kernel(B, T, H, Wd)(bins.reshape(-1))
  halves = 32 // B
  hist = partials.reshape(B, halves, NBINS, TPADDED)

  odim = W.shape[0]
  out = pl.pallas_call(
      functools.partial(_dense_body, t=T),
      out_shape=jax.ShapeDtypeStruct((B, T, odim), jnp.float32),
      grid=(B,),
      in_specs=[
          pl.BlockSpec((1, halves, NBINS, TPADDED), lambda i: (i, 0, 0, 0)),
          pl.BlockSpec((odim, LOOKUP), lambda i: (0, 0)),
          pl.BlockSpec((1, odim), lambda i: (0, 0)),
      ],
      out_specs=pl.BlockSpec((1, T, odim), lambda i: (i, 0, 0)),
  )(hist, W[:, ::-1], b.reshape(1, odim))
  return out
